# baseline (speedup 1.0000x reference)
import jax
import jax.numpy as jnp
from jax.experimental import pallas as pl
from jax.experimental.pallas import tpu as pltpu

_CHUNKS = 16


def _tc_broadcast(B: int, P: int, D: int, dtype, table):
    nch = _CHUNKS if P % _CHUNKS == 0 else 1
    rows_c = P // nch

    def body(emb_hbm, out_hbm, rows_vmem, in_sems, out_sem):
        reads = [
            pltpu.make_async_copy(
                emb_hbm.at[pl.ds(i * rows_c, rows_c), :],
                rows_vmem.at[pl.ds(i * rows_c, rows_c), :],
                in_sems.at[i],
            )
            for i in range(nch)
        ]
        for r in reads:
            r.start()
        writes = []
        for i in range(nch):
            reads[i].wait()
            for b in range(B):
                cp = pltpu.make_async_copy(
                    rows_vmem.at[pl.ds(i * rows_c, rows_c), :],
                    out_hbm.at[b, pl.ds(i * rows_c, rows_c), :],
                    out_sem,
                )
                cp.start()
                writes.append(cp)
        for cp in writes:
            cp.wait()

    return pl.pallas_call(
        body,
        in_specs=[pl.BlockSpec(memory_space=pl.ANY)],
        out_specs=pl.BlockSpec(memory_space=pl.ANY),
        out_shape=jax.ShapeDtypeStruct((B, P, D), dtype),
        scratch_shapes=[
            pltpu.VMEM((P, D), dtype),
            pltpu.SemaphoreType.DMA((nch,)),
            pltpu.SemaphoreType.DMA,
        ],
    )(table)


def kernel(x, pos_embed):
    B, C, H, W = x.shape
    P = H * W
    D = pos_embed.shape[1]
    return _tc_broadcast(B, P, D, pos_embed.dtype, pos_embed)


# speedup vs baseline: 1.0217x; 1.0217x over previous
import jax
import jax.numpy as jnp
from jax.experimental import pallas as pl
from jax.experimental.pallas import tpu as pltpu

_CHUNKS = 8


def _broadcast_rows(B: int, P: int, D: int, dtype, table):
    nch = _CHUNKS if P % _CHUNKS == 0 else 1
    rows_c = P // nch

    def body(emb_hbm, out_hbm, rows_vmem, in_sems, out_sem):
        reads = [
            pltpu.make_async_copy(
                emb_hbm.at[pl.ds(i * rows_c, rows_c), :],
                rows_vmem.at[pl.ds(i * rows_c, rows_c), :],
                in_sems.at[i],
            )
            for i in range(nch)
        ]
        for r in reads:
            r.start()
        writes = []
        for i in range(nch):
            reads[i].wait()
            for b in range(B):
                cp = pltpu.make_async_copy(
                    rows_vmem.at[pl.ds(i * rows_c, rows_c), :],
                    out_hbm.at[b, pl.ds(i * rows_c, rows_c), :],
                    out_sem,
                )
                cp.start()
                writes.append(cp)
        for cp in writes:
            cp.wait()

    return pl.pallas_call(
        body,
        in_specs=[pl.BlockSpec(memory_space=pl.ANY)],
        out_specs=pl.BlockSpec(memory_space=pl.ANY),
        out_shape=jax.ShapeDtypeStruct((B, P, D), dtype),
        scratch_shapes=[
            pltpu.VMEM((P, D), dtype),
            pltpu.SemaphoreType.DMA((nch,)),
            pltpu.SemaphoreType.DMA,
        ],
    )(table)


def kernel(x, pos_embed):
    B, C, H, W = x.shape
    P = H * W
    D = pos_embed.shape[1]
    return _broadcast_rows(B, P, D, pos_embed.dtype, pos_embed)
